# baseline (speedup 1.0000x reference)
import jax
import jax.numpy as jnp
from jax import lax
from jax.experimental import pallas as pl
from jax.experimental.pallas import tpu as pltpu
from jax.experimental.pallas import tpu_sc as plsc

_N_PAIRS = 16384
_N_ROWS = 100000
_D = 64
_NC = 2
_NS = 16
_NW = _NC * _NS
_PER_W = _N_PAIRS // _NW
_CHUNK = 128
_NCHUNK = _PER_W // _CHUNK
_L = 16
_NSIDE_CHUNKS = 2 * _NCHUNK


def _sc_scores_body(u_pos_t, i_pos_t, u_neg_t, i_neg_t,
                    uidx_pos, iidx_pos, uidx_neg, iidx_neg,
                    pos_out, neg_out,
                    uidx_v, iidx_v,
                    urows_v, irows_v, scores_v, sem0, sem1):
    wid = lax.axis_index("s") * _NC + lax.axis_index("c")
    base = wid * _PER_W
    pltpu.sync_copy(uidx_pos.at[pl.ds(base, _PER_W)], uidx_v.at[0])
    pltpu.sync_copy(iidx_pos.at[pl.ds(base, _PER_W)], iidx_v.at[0])
    pltpu.sync_copy(uidx_neg.at[pl.ds(base, _PER_W)], uidx_v.at[1])
    pltpu.sync_copy(iidx_neg.at[pl.ds(base, _PER_W)], iidx_v.at[1])

    sems = (sem0, sem1)
    sides = ((u_pos_t, i_pos_t, pos_out), (u_neg_t, i_neg_t, neg_out))
    lane_iota = lax.iota(jnp.int32, _L)
    perms = {s: lane_iota ^ s for s in (8, 4, 2, 1)}

    def issue(c):
        side, j = c // _NCHUNK, c % _NCHUNK
        slot = c % 2
        u_t, i_t, _ = sides[side]
        urows = urows_v.at[slot]
        irows = irows_v.at[slot]
        sem = sems[slot]

        def dma_group(g, carry):
            uvec = uidx_v[side, pl.ds(j * _CHUNK + g * _L, _L)]
            ivec = iidx_v[side, pl.ds(j * _CHUNK + g * _L, _L)]
            for l in range(_L):
                p = g * _L + l
                msk = lane_iota == l
                ur = lax.reduce_sum(jnp.where(msk, uvec, 0), axes=(0,))
                ir = lax.reduce_sum(jnp.where(msk, ivec, 0), axes=(0,))
                pltpu.async_copy(u_t.at[pl.ds(ur, 1)],
                                 urows.at[pl.ds(p, 1)], sem)
                pltpu.async_copy(i_t.at[pl.ds(ir, 1)],
                                 irows.at[pl.ds(p, 1)], sem)
            return carry

        lax.fori_loop(0, _CHUNK // _L, dma_group, 0)

    def drain(c):
        slot = c % 2
        dummy = u_pos_t.at[pl.ds(0, _CHUNK)]
        pltpu.make_async_copy(dummy, urows_v.at[slot], sems[slot]).wait()
        pltpu.make_async_copy(dummy, irows_v.at[slot], sems[slot]).wait()

    issue(0)
    issue(1)
    for c in range(_NSIDE_CHUNKS):
        side, j = c // _NCHUNK, c % _NCHUNK
        slot = c % 2
        out = sides[side][2]
        drain(c)
        urows = urows_v.at[slot]
        irows = irows_v.at[slot]

        def group_body(g, carry, urows=urows, irows=irows):
            for l in range(_L):
                p = g * _L + l
                rows16 = jnp.full((_L,), p, jnp.int32)
                acc = None
                for kc in range(_D // _L):
                    ccol = kc * _L + lane_iota
                    uvk = plsc.bitcast(
                        plsc.load_gather(urows, [rows16, ccol]),
                        jnp.float32)
                    ivk = plsc.bitcast(
                        plsc.load_gather(irows, [rows16, ccol]),
                        jnp.float32)
                    prod = uvk * ivk
                    acc = prod if acc is None else acc + prod
                for s in (8, 4, 2, 1):
                    acc = acc + acc[perms[s]]
                plsc.store_scatter(scores_v, [rows16], acc,
                                   mask=lane_iota == l)
            return carry

        lax.fori_loop(0, _CHUNK // _L, group_body, 0)
        pltpu.sync_copy(scores_v, out.at[pl.ds(base + j * _CHUNK, _CHUNK)])
        if c + 2 < _NSIDE_CHUNKS:
            issue(c + 2)


def _sc_scores(u_pos_t, i_pos_t, u_neg_t, i_neg_t, idx_arrays):
    mesh = plsc.VectorSubcoreMesh(core_axis_name="c", subcore_axis_name="s",
                                  num_cores=_NC, num_subcores=_NS)
    fn = pl.kernel(
        _sc_scores_body,
        out_type=[jax.ShapeDtypeStruct((_N_PAIRS,), jnp.float32),
                  jax.ShapeDtypeStruct((_N_PAIRS,), jnp.float32)],
        mesh=mesh,
        compiler_params=pltpu.CompilerParams(needs_layout_passes=False,
                                             use_tc_tiling_on_sc=True),
        scratch_types=[
            pltpu.VMEM((2, _PER_W), jnp.int32),
            pltpu.VMEM((2, _PER_W), jnp.int32),
            pltpu.VMEM((2, _CHUNK, _D), jnp.int32),
            pltpu.VMEM((2, _CHUNK, _D), jnp.int32),
            pltpu.VMEM((_CHUNK,), jnp.float32),
            pltpu.SemaphoreType.DMA,
            pltpu.SemaphoreType.DMA,
        ],
    )
    return fn(u_pos_t, i_pos_t, u_neg_t, i_neg_t, *idx_arrays)


def _loss_body(pos_ref, neg_ref, out_ref):
    pos = pos_ref[...]
    neg = neg_ref[...]
    total = jnp.sum(jax.nn.log_sigmoid(pos) + jax.nn.log_sigmoid(-neg))
    out_ref[0, 0] = -total / _N_PAIRS


def _loss(pos_scores, neg_scores):
    p = pos_scores.reshape(_N_PAIRS // 128, 128)
    n = neg_scores.reshape(_N_PAIRS // 128, 128)
    out = pl.pallas_call(
        _loss_body,
        out_shape=jax.ShapeDtypeStruct((1, 1), jnp.float32),
        out_specs=pl.BlockSpec(memory_space=pltpu.SMEM),
    )(p, n)
    return out[0, 0]


def kernel(user_emb_pos, item_emb_pos, user_emb_neg, item_emb_neg,
           positive_pairs, negative_pairs):
    t3 = lambda t: lax.bitcast_convert_type(t, jnp.int32)
    pos_s, neg_s = _sc_scores(
        t3(user_emb_pos), t3(item_emb_pos),
        t3(user_emb_neg), t3(item_emb_neg),
        (positive_pairs[:, 0].astype(jnp.int32),
         positive_pairs[:, 1].astype(jnp.int32),
         negative_pairs[:, 0].astype(jnp.int32),
         negative_pairs[:, 1].astype(jnp.int32)))
    return _loss(pos_s, neg_s)


# speedup vs baseline: 1.6224x; 1.6224x over previous
import jax
import jax.numpy as jnp
from jax import lax
from jax.experimental import pallas as pl
from jax.experimental.pallas import tpu as pltpu
from jax.experimental.pallas import tpu_sc as plsc

_N_PAIRS = 16384
_N_ROWS = 100000
_D = 64
_NC = 2
_NS = 16
_NW = _NC * _NS
_PER_W = _N_PAIRS // _NW
_CHUNK = 128
_NCHUNK = _PER_W // _CHUNK
_L = 16
_NSIDE_CHUNKS = 2 * _NCHUNK


def _sc_scores_body(u_pos_t, i_pos_t, u_neg_t, i_neg_t,
                    uidx_pos, iidx_pos, uidx_neg, iidx_neg,
                    pos_out, neg_out,
                    uidx_v, iidx_v,
                    urows_v, irows_v, scores_v, sem0, sem1):
    wid = lax.axis_index("s") * _NC + lax.axis_index("c")
    base = wid * _PER_W
    pltpu.sync_copy(uidx_pos.at[pl.ds(base, _PER_W)], uidx_v.at[0])
    pltpu.sync_copy(iidx_pos.at[pl.ds(base, _PER_W)], iidx_v.at[0])
    pltpu.sync_copy(uidx_neg.at[pl.ds(base, _PER_W)], uidx_v.at[1])
    pltpu.sync_copy(iidx_neg.at[pl.ds(base, _PER_W)], iidx_v.at[1])

    sems = (sem0, sem1)
    sides = ((u_pos_t, i_pos_t, pos_out), (u_neg_t, i_neg_t, neg_out))
    lane_iota = lax.iota(jnp.int32, _L)
    perms = {s: lane_iota ^ s for s in (8, 4, 2, 1)}

    def issue(c):
        side, j = c // _NCHUNK, c % _NCHUNK
        slot = c % 2
        u_t, i_t, _ = sides[side]
        urows = urows_v.at[slot]
        irows = irows_v.at[slot]
        sem = sems[slot]

        def dma_group(g, carry):
            uvec = uidx_v[side, pl.ds(j * _CHUNK + g * _L, _L)]
            ivec = iidx_v[side, pl.ds(j * _CHUNK + g * _L, _L)]
            for l in range(_L):
                p = g * _L + l
                msk = lane_iota == l
                ur = lax.reduce_sum(jnp.where(msk, uvec, 0), axes=(0,))
                ir = lax.reduce_sum(jnp.where(msk, ivec, 0), axes=(0,))
                pltpu.async_copy(u_t.at[ur // 8, pl.ds(ur % 8, 1)],
                                 urows.at[p // 8, pl.ds(p % 8, 1)], sem)
                pltpu.async_copy(i_t.at[ir // 8, pl.ds(ir % 8, 1)],
                                 irows.at[p // 8, pl.ds(p % 8, 1)], sem)
            return carry

        lax.fori_loop(0, _CHUNK // _L, dma_group, 0)

    def drain(c):
        slot = c % 2
        dummy = u_pos_t.at[pl.ds(0, _CHUNK // 8)]
        pltpu.make_async_copy(dummy, urows_v.at[slot], sems[slot]).wait()
        pltpu.make_async_copy(dummy, irows_v.at[slot], sems[slot]).wait()

    issue(0)
    issue(1)
    for c in range(_NSIDE_CHUNKS):
        side, j = c // _NCHUNK, c % _NCHUNK
        slot = c % 2
        out = sides[side][2]
        drain(c)
        urows = urows_v.at[slot]
        irows = irows_v.at[slot]

        def group_body(g, carry, urows=urows, irows=irows):
            for l in range(_L):
                p = g * _L + l
                rows16 = jnp.full((_L,), p, jnp.int32)
                r3a = jnp.full((_L,), p // 8, jnp.int32)
                r3b = jnp.full((_L,), p % 8, jnp.int32)
                acc = None
                for kc in range(_D // _L):
                    ccol = kc * _L + lane_iota
                    uvk = plsc.load_gather(urows, [r3a, r3b, ccol])
                    ivk = plsc.load_gather(irows, [r3a, r3b, ccol])
                    prod = uvk * ivk
                    acc = prod if acc is None else acc + prod
                for s in (8, 4, 2, 1):
                    acc = acc + acc[perms[s]]
                plsc.store_scatter(scores_v, [rows16], acc,
                                   mask=lane_iota == l)
            return carry

        lax.fori_loop(0, _CHUNK // _L, group_body, 0)
        pltpu.sync_copy(scores_v, out.at[pl.ds(base + j * _CHUNK, _CHUNK)])
        if c + 2 < _NSIDE_CHUNKS:
            issue(c + 2)


def _sc_scores(u_pos_t, i_pos_t, u_neg_t, i_neg_t, idx_arrays):
    mesh = plsc.VectorSubcoreMesh(core_axis_name="c", subcore_axis_name="s",
                                  num_cores=_NC, num_subcores=_NS)
    fn = pl.kernel(
        _sc_scores_body,
        out_type=[jax.ShapeDtypeStruct((_N_PAIRS,), jnp.float32),
                  jax.ShapeDtypeStruct((_N_PAIRS,), jnp.float32)],
        mesh=mesh,
        compiler_params=pltpu.CompilerParams(needs_layout_passes=False,
                                             use_tc_tiling_on_sc=True),
        scratch_types=[
            pltpu.VMEM((2, _PER_W), jnp.int32),
            pltpu.VMEM((2, _PER_W), jnp.int32),
            pltpu.VMEM((2, _CHUNK // 8, 8, _D), jnp.float32),
            pltpu.VMEM((2, _CHUNK // 8, 8, _D), jnp.float32),
            pltpu.VMEM((_CHUNK,), jnp.float32),
            pltpu.SemaphoreType.DMA,
            pltpu.SemaphoreType.DMA,
        ],
    )
    return fn(u_pos_t, i_pos_t, u_neg_t, i_neg_t, *idx_arrays)


def _loss_body(pos_ref, neg_ref, out_ref):
    pos = pos_ref[...]
    neg = neg_ref[...]
    total = jnp.sum(jax.nn.log_sigmoid(pos) + jax.nn.log_sigmoid(-neg))
    out_ref[0, 0] = -total / _N_PAIRS


def _loss(pos_scores, neg_scores):
    p = pos_scores.reshape(_N_PAIRS // 128, 128)
    n = neg_scores.reshape(_N_PAIRS // 128, 128)
    out = pl.pallas_call(
        _loss_body,
        out_shape=jax.ShapeDtypeStruct((1, 1), jnp.float32),
        out_specs=pl.BlockSpec(memory_space=pltpu.SMEM),
    )(p, n)
    return out[0, 0]


def kernel(user_emb_pos, item_emb_pos, user_emb_neg, item_emb_neg,
           positive_pairs, negative_pairs):
    t3 = lambda t: t.reshape(_N_ROWS // 8, 8, _D)
    pos_s, neg_s = _sc_scores(
        t3(user_emb_pos), t3(item_emb_pos),
        t3(user_emb_neg), t3(item_emb_neg),
        (positive_pairs[:, 0].astype(jnp.int32),
         positive_pairs[:, 1].astype(jnp.int32),
         negative_pairs[:, 0].astype(jnp.int32),
         negative_pairs[:, 1].astype(jnp.int32)))
    return _loss(pos_s, neg_s)
